# baseline (speedup 1.0000x reference)
import jax
import jax.numpy as jnp
from jax import lax
from jax.experimental import pallas as pl
from jax.experimental.pallas import tpu as pltpu
from jax.experimental.pallas import tpu_sc as plsc

N = 10000
E = 320000
D_FEAT = 128
D_LABEL = 16
H = 128
OUT = 16

NC = 2
NS = 16
B = 640
NB = 32
T = 32
ACC_ROWS = 10240
DUMMY = N
ZCH = 10
COPY_ROWS = ACC_ROWS // NS
QW = 32
NQ = 4
TCOPY = N // NS


def _spmm_body(tf0, tf1, tf2, tf3, tl0, tl1, tl2, tl3, srcs, dsts, zsrc,
               of10, of11, of12, of13, of20, of21, of22, of23,
               ol10, ol11, ol12, ol13, ol20, ol21, ol22, ol23,
               isrc, idst, buf0, buf1, zbuf, tbl, acc,
               sem0, sem1, ssem0, ssem1):
    cid = lax.axis_index("c")
    sid = lax.axis_index("s")
    pltpu.sync_copy(zsrc, zbuf)

    def run(tables, outs):
        for es in range(2):
            pltpu.sync_copy(srcs.at[es, pl.ds(sid * T, T)], isrc)
            pltpu.sync_copy(dsts.at[es, pl.ds(sid * T, T)], idst)
            for q in range(NQ):
                pltpu.sync_copy(tables[q].at[pl.ds(sid * TCOPY, TCOPY)],
                                tbl.at[pl.ds(sid * TCOPY, TCOPY)])
                base = 0
                for k in range(ZCH):
                    pltpu.sync_copy(
                        zbuf, acc.at[pl.ds(sid * (64 * ZCH) + k * 64, 64)])
                plsc.subcore_barrier()
                pltpu.async_copy(tbl.at[isrc.at[base + 0]], buf0, sem0)
                pltpu.async_copy(tbl.at[isrc.at[base + 1]], buf1, sem1)

                def it(jj, carry):
                    j = base + 2 * jj
                    pltpu.make_async_copy(tbl.at[isrc.at[j]], buf0, sem0).wait()
                    pltpu.async_copy(buf0, acc.at[idst.at[j]], ssem0, add=True)
                    pltpu.make_async_copy(tbl.at[isrc.at[j + 1]], buf1, sem1).wait()
                    pltpu.async_copy(buf1, acc.at[idst.at[j + 1]], ssem1, add=True)
                    pltpu.make_async_copy(buf0, acc.at[idst.at[j]], ssem0).wait()
                    pltpu.async_copy(tbl.at[isrc.at[j + 2]], buf0, sem0)
                    pltpu.make_async_copy(buf1, acc.at[idst.at[j + 1]], ssem1).wait()
                    pltpu.async_copy(tbl.at[isrc.at[j + 3]], buf1, sem1)
                    return carry

                lax.fori_loop(0, (T - 2) // 2, it, 0)
                pltpu.make_async_copy(tbl.at[isrc.at[base + T - 2]], buf0, sem0).wait()
                pltpu.async_copy(buf0, acc.at[idst.at[base + T - 2]], ssem0, add=True)
                pltpu.make_async_copy(tbl.at[isrc.at[base + T - 1]], buf1, sem1).wait()
                pltpu.async_copy(buf1, acc.at[idst.at[base + T - 1]], ssem1, add=True)
                pltpu.make_async_copy(buf0, acc.at[idst.at[base + T - 2]], ssem0).wait()
                pltpu.make_async_copy(buf1, acc.at[idst.at[base + T - 1]], ssem1).wait()
                plsc.subcore_barrier()
                pltpu.sync_copy(acc.at[pl.ds(sid * COPY_ROWS, COPY_ROWS)],
                                outs[es][q].at[pl.ds(sid * COPY_ROWS,
                                                     COPY_ROWS)])
            plsc.subcore_barrier()

    @pl.when(cid == 0)
    def _():
        run((tf0, tf1, tf2, tf3),
            ((of10, of11, of12, of13), (of20, of21, of22, of23)))

    @pl.when(cid == 1)
    def _():
        run((tl0, tl1, tl2, tl3),
            ((ol10, ol11, ol12, ol13), (ol20, ol21, ol22, ol23)))


def _make_spmm():
    if _SPMM_CACHE:
        return _SPMM_CACHE[0]
    f = pl.kernel(
        _spmm_body,
        out_type=[jax.ShapeDtypeStruct((ACC_ROWS, QW), jnp.float32)] * 16,
        mesh=plsc.VectorSubcoreMesh(core_axis_name="c", subcore_axis_name="s",
                                    num_cores=NC, num_subcores=NS),
        compiler_params=pltpu.CompilerParams(use_tc_tiling_on_sc=False),
        scratch_types=[
            pltpu.VMEM((T, B), jnp.int32),
            pltpu.VMEM((T, B), jnp.int32),
            pltpu.VMEM((B, QW), jnp.float32),
            pltpu.VMEM((B, QW), jnp.float32),
            pltpu.VMEM((64, QW), jnp.float32),
            pltpu.VMEM_SHARED((N, QW), jnp.float32),
            pltpu.VMEM_SHARED((ACC_ROWS, QW), jnp.float32),
            pltpu.SemaphoreType.DMA,
            pltpu.SemaphoreType.DMA,
            pltpu.SemaphoreType.DMA,
            pltpu.SemaphoreType.DMA,
        ],
    )
    _SPMM_CACHE.append(f)
    return f


_SPMM_CACHE = []


def _prep_edges(e):
    pad = NS * NB * B - E
    src = jnp.pad(e[0], (0, pad)).reshape(NS, NB, B)
    dst = jnp.pad(e[1], (0, pad), constant_values=DUMMY).reshape(NS, NB, B)
    src = jnp.concatenate([src, jnp.zeros((NS, T - NB, B), jnp.int32)], axis=1)
    dst = jnp.concatenate([dst, jnp.full((NS, T - NB, B), DUMMY, jnp.int32)],
                          axis=1)
    return src.reshape(NS * T, B), dst.reshape(NS * T, B)


RB = 1000


def _proj_body(xf, xl, wf, wl, h0f, h0l):
    h0f[...] = jnp.dot(xf[...], wf[...], preferred_element_type=jnp.float32)
    h0l[...] = jnp.dot(xl[...], wl[...], preferred_element_type=jnp.float32)


_proj = pl.pallas_call(
    _proj_body,
    grid=(N // RB,),
    in_specs=[
        pl.BlockSpec((RB, D_FEAT), lambda i: (i, 0)),
        pl.BlockSpec((RB, D_LABEL), lambda i: (i, 0)),
        pl.BlockSpec((D_FEAT, H), lambda i: (0, 0)),
        pl.BlockSpec((D_LABEL, H), lambda i: (0, 0)),
    ],
    out_specs=[
        pl.BlockSpec((RB, H), lambda i: (i, 0)),
        pl.BlockSpec((RB, H), lambda i: (i, 0)),
    ],
    out_shape=[jax.ShapeDtypeStruct((N, H), jnp.float32)] * 2,
)


def _gate_body(h0f, f10, f11, f12, f13, f20, f21, f22, f23,
               h0l, l10, l11, l12, l13, l20, l21, l22, l23,
               wg1, bg1, wg2r, bg2, wout, bout, out, gate):
    hf = jnp.concatenate(
        [h0f[...], f10[...], f11[...], f12[...], f13[...],
         f20[...], f21[...], f22[...], f23[...]], axis=1)
    hl = jnp.concatenate(
        [h0l[...], l10[...], l11[...], l12[...], l13[...],
         l20[...], l21[...], l22[...], l23[...]], axis=1)
    gi = jnp.concatenate([hf, hl], axis=1)
    t = jnp.maximum(
        jnp.dot(gi, wg1[...], preferred_element_type=jnp.float32) + bg1[...], 0.0)
    g = jax.nn.sigmoid(jnp.sum(t * wg2r[...], axis=1, keepdims=True) + bg2[...])
    fused = g * hf + (1.0 - g) * hl
    out[...] = jnp.dot(fused, wout[...], preferred_element_type=jnp.float32) + bout[...]
    gate[...] = jnp.broadcast_to(g, (RB, 3 * H))


_gate = pl.pallas_call(
    _gate_body,
    grid=(N // RB,),
    in_specs=[pl.BlockSpec((RB, H), lambda i: (i, 0))]
    + [pl.BlockSpec((RB, QW), lambda i: (i, 0))] * 8
    + [pl.BlockSpec((RB, H), lambda i: (i, 0))]
    + [pl.BlockSpec((RB, QW), lambda i: (i, 0))] * 8 + [
        pl.BlockSpec((6 * H, H), lambda i: (0, 0)),
        pl.BlockSpec((1, H), lambda i: (0, 0)),
        pl.BlockSpec((1, H), lambda i: (0, 0)),
        pl.BlockSpec((1, 1), lambda i: (0, 0)),
        pl.BlockSpec((3 * H, OUT), lambda i: (0, 0)),
        pl.BlockSpec((1, OUT), lambda i: (0, 0)),
    ],
    out_specs=[
        pl.BlockSpec((RB, OUT), lambda i: (i, 0)),
        pl.BlockSpec((RB, 3 * H), lambda i: (i, 0)),
    ],
    out_shape=[
        jax.ShapeDtypeStruct((N, OUT), jnp.float32),
        jax.ShapeDtypeStruct((N, 3 * H), jnp.float32),
    ],
)


def kernel(x_feat, x_label, edge_index1, edge_index2,
           W_feat, W_label, Wg1, bg1, Wg2, bg2, Wout, bout):
    h0f, h0l = _proj(x_feat, x_label, W_feat, W_label)
    s1, d1 = _prep_edges(edge_index1)
    s2, d2 = _prep_edges(edge_index2)
    srcs = jnp.stack([s1, s2])
    dsts = jnp.stack([d1, d2])
    fq = [h0f[:, i * QW:(i + 1) * QW] for i in range(NQ)]
    lq = [h0l[:, i * QW:(i + 1) * QW] for i in range(NQ)]
    outs = _make_spmm()(
        *fq, *lq, srcs, dsts, jnp.zeros((64, QW), jnp.float32))
    out, gate = _gate(
        h0f, *outs[0:8], h0l, *outs[8:16],
        Wg1, bg1.reshape(1, H), Wg2.reshape(1, H), bg2.reshape(1, 1),
        Wout, bout.reshape(1, OUT))
    return (out, gate)


# speedup vs baseline: 1.0809x; 1.0809x over previous
import jax
import jax.numpy as jnp
from jax import lax
from jax.experimental import pallas as pl
from jax.experimental.pallas import tpu as pltpu
from jax.experimental.pallas import tpu_sc as plsc

N = 10000
E = 320000
D_FEAT = 128
D_LABEL = 16
H = 128
OUT = 16

NC = 2
NS = 16
B = 500
NB = 40
T = 40
ACC_ROWS = 10240
DUMMY = N
ZCH = 10
COPY_ROWS = ACC_ROWS // NS
QW = 32
NQ = 4
TCOPY = N // NS


def _spmm_body(tf0, tf1, tf2, tf3, tl0, tl1, tl2, tl3, srcs, dsts, zsrc,
               of10, of11, of12, of13, of20, of21, of22, of23,
               ol10, ol11, ol12, ol13, ol20, ol21, ol22, ol23,
               isrc, idst, buf0, buf1, zbuf, tbl, acc,
               sem0, sem1, ssem0, ssem1):
    cid = lax.axis_index("c")
    sid = lax.axis_index("s")
    pltpu.sync_copy(zsrc, zbuf)

    def run(tables, outs):
        for es in range(2):
            pltpu.sync_copy(srcs.at[es, pl.ds(sid * T, T)], isrc)
            pltpu.sync_copy(dsts.at[es, pl.ds(sid * T, T)], idst)
            for q in range(NQ):
                pltpu.sync_copy(tables[q].at[pl.ds(sid * TCOPY, TCOPY)],
                                tbl.at[pl.ds(sid * TCOPY, TCOPY)])
                base = 0
                for k in range(ZCH):
                    pltpu.sync_copy(
                        zbuf, acc.at[pl.ds(sid * (64 * ZCH) + k * 64, 64)])
                plsc.subcore_barrier()
                pltpu.async_copy(tbl.at[isrc.at[base + 0]], buf0, sem0)
                pltpu.async_copy(tbl.at[isrc.at[base + 1]], buf1, sem1)

                def it(jj, carry):
                    j = base + 2 * jj
                    pltpu.make_async_copy(tbl.at[isrc.at[j]], buf0, sem0).wait()
                    pltpu.async_copy(buf0, acc.at[idst.at[j]], ssem0, add=True)
                    pltpu.make_async_copy(tbl.at[isrc.at[j + 1]], buf1, sem1).wait()
                    pltpu.async_copy(buf1, acc.at[idst.at[j + 1]], ssem1, add=True)
                    pltpu.make_async_copy(buf0, acc.at[idst.at[j]], ssem0).wait()
                    pltpu.async_copy(tbl.at[isrc.at[j + 2]], buf0, sem0)
                    pltpu.make_async_copy(buf1, acc.at[idst.at[j + 1]], ssem1).wait()
                    pltpu.async_copy(tbl.at[isrc.at[j + 3]], buf1, sem1)
                    return carry

                lax.fori_loop(0, (T - 2) // 2, it, 0)
                pltpu.make_async_copy(tbl.at[isrc.at[base + T - 2]], buf0, sem0).wait()
                pltpu.async_copy(buf0, acc.at[idst.at[base + T - 2]], ssem0, add=True)
                pltpu.make_async_copy(tbl.at[isrc.at[base + T - 1]], buf1, sem1).wait()
                pltpu.async_copy(buf1, acc.at[idst.at[base + T - 1]], ssem1, add=True)
                pltpu.make_async_copy(buf0, acc.at[idst.at[base + T - 2]], ssem0).wait()
                pltpu.make_async_copy(buf1, acc.at[idst.at[base + T - 1]], ssem1).wait()
                plsc.subcore_barrier()
                pltpu.sync_copy(acc.at[pl.ds(sid * COPY_ROWS, COPY_ROWS)],
                                outs[es][q].at[pl.ds(sid * COPY_ROWS,
                                                     COPY_ROWS)])
            plsc.subcore_barrier()

    @pl.when(cid == 0)
    def _():
        run((tf0, tf1, tf2, tf3),
            ((of10, of11, of12, of13), (of20, of21, of22, of23)))

    @pl.when(cid == 1)
    def _():
        run((tl0, tl1, tl2, tl3),
            ((ol10, ol11, ol12, ol13), (ol20, ol21, ol22, ol23)))


def _make_spmm():
    if _SPMM_CACHE:
        return _SPMM_CACHE[0]
    f = pl.kernel(
        _spmm_body,
        out_type=[jax.ShapeDtypeStruct((ACC_ROWS, QW), jnp.float32)] * 16,
        mesh=plsc.VectorSubcoreMesh(core_axis_name="c", subcore_axis_name="s",
                                    num_cores=NC, num_subcores=NS),
        compiler_params=pltpu.CompilerParams(use_tc_tiling_on_sc=False),
        scratch_types=[
            pltpu.VMEM((T, B), jnp.int32),
            pltpu.VMEM((T, B), jnp.int32),
            pltpu.VMEM((B, QW), jnp.float32),
            pltpu.VMEM((B, QW), jnp.float32),
            pltpu.VMEM((64, QW), jnp.float32),
            pltpu.VMEM_SHARED((N, QW), jnp.float32),
            pltpu.VMEM_SHARED((ACC_ROWS, QW), jnp.float32),
            pltpu.SemaphoreType.DMA,
            pltpu.SemaphoreType.DMA,
            pltpu.SemaphoreType.DMA,
            pltpu.SemaphoreType.DMA,
        ],
    )
    _SPMM_CACHE.append(f)
    return f


_SPMM_CACHE = []


def _prep_edges(e):
    pad = NS * NB * B - E
    src = jnp.pad(e[0], (0, pad)).reshape(NS, NB, B)
    dst = jnp.pad(e[1], (0, pad), constant_values=DUMMY).reshape(NS, NB, B)
    src = jnp.concatenate([src, jnp.zeros((NS, T - NB, B), jnp.int32)], axis=1)
    dst = jnp.concatenate([dst, jnp.full((NS, T - NB, B), DUMMY, jnp.int32)],
                          axis=1)
    return src.reshape(NS * T, B), dst.reshape(NS * T, B)


RB = 1000


def _proj_body(xf, xl, wf, wl, h0f, h0l):
    h0f[...] = jnp.dot(xf[...], wf[...], preferred_element_type=jnp.float32)
    h0l[...] = jnp.dot(xl[...], wl[...], preferred_element_type=jnp.float32)


_proj = pl.pallas_call(
    _proj_body,
    grid=(N // RB,),
    in_specs=[
        pl.BlockSpec((RB, D_FEAT), lambda i: (i, 0)),
        pl.BlockSpec((RB, D_LABEL), lambda i: (i, 0)),
        pl.BlockSpec((D_FEAT, H), lambda i: (0, 0)),
        pl.BlockSpec((D_LABEL, H), lambda i: (0, 0)),
    ],
    out_specs=[
        pl.BlockSpec((RB, H), lambda i: (i, 0)),
        pl.BlockSpec((RB, H), lambda i: (i, 0)),
    ],
    out_shape=[jax.ShapeDtypeStruct((N, H), jnp.float32)] * 2,
)


def _gate_body(h0f, f10, f11, f12, f13, f20, f21, f22, f23,
               h0l, l10, l11, l12, l13, l20, l21, l22, l23,
               wg1, bg1, wg2r, bg2, wout, bout, out, gate):
    hf = jnp.concatenate(
        [h0f[...], f10[...], f11[...], f12[...], f13[...],
         f20[...], f21[...], f22[...], f23[...]], axis=1)
    hl = jnp.concatenate(
        [h0l[...], l10[...], l11[...], l12[...], l13[...],
         l20[...], l21[...], l22[...], l23[...]], axis=1)
    gi = jnp.concatenate([hf, hl], axis=1)
    t = jnp.maximum(
        jnp.dot(gi, wg1[...], preferred_element_type=jnp.float32) + bg1[...], 0.0)
    g = jax.nn.sigmoid(jnp.sum(t * wg2r[...], axis=1, keepdims=True) + bg2[...])
    fused = g * hf + (1.0 - g) * hl
    out[...] = jnp.dot(fused, wout[...], preferred_element_type=jnp.float32) + bout[...]
    gate[...] = jnp.broadcast_to(g, (RB, 3 * H))


_gate = pl.pallas_call(
    _gate_body,
    grid=(N // RB,),
    in_specs=[pl.BlockSpec((RB, H), lambda i: (i, 0))]
    + [pl.BlockSpec((RB, QW), lambda i: (i, 0))] * 8
    + [pl.BlockSpec((RB, H), lambda i: (i, 0))]
    + [pl.BlockSpec((RB, QW), lambda i: (i, 0))] * 8 + [
        pl.BlockSpec((6 * H, H), lambda i: (0, 0)),
        pl.BlockSpec((1, H), lambda i: (0, 0)),
        pl.BlockSpec((1, H), lambda i: (0, 0)),
        pl.BlockSpec((1, 1), lambda i: (0, 0)),
        pl.BlockSpec((3 * H, OUT), lambda i: (0, 0)),
        pl.BlockSpec((1, OUT), lambda i: (0, 0)),
    ],
    out_specs=[
        pl.BlockSpec((RB, OUT), lambda i: (i, 0)),
        pl.BlockSpec((RB, 3 * H), lambda i: (i, 0)),
    ],
    out_shape=[
        jax.ShapeDtypeStruct((N, OUT), jnp.float32),
        jax.ShapeDtypeStruct((N, 3 * H), jnp.float32),
    ],
)


def kernel(x_feat, x_label, edge_index1, edge_index2,
           W_feat, W_label, Wg1, bg1, Wg2, bg2, Wout, bout):
    h0f, h0l = _proj(x_feat, x_label, W_feat, W_label)
    s1, d1 = _prep_edges(edge_index1)
    s2, d2 = _prep_edges(edge_index2)
    srcs = jnp.stack([s1, s2])
    dsts = jnp.stack([d1, d2])
    fq = [h0f[:, i * QW:(i + 1) * QW] for i in range(NQ)]
    lq = [h0l[:, i * QW:(i + 1) * QW] for i in range(NQ)]
    outs = _make_spmm()(
        *fq, *lq, srcs, dsts, jnp.zeros((64, QW), jnp.float32))
    out, gate = _gate(
        h0f, *outs[0:8], h0l, *outs[8:16],
        Wg1, bg1.reshape(1, H), Wg2.reshape(1, H), bg2.reshape(1, 1),
        Wout, bout.reshape(1, OUT))
    return (out, gate)


# speedup vs baseline: 1.0876x; 1.0061x over previous
import jax
import jax.numpy as jnp
from jax import lax
from jax.experimental import pallas as pl
from jax.experimental.pallas import tpu as pltpu
from jax.experimental.pallas import tpu_sc as plsc

N = 10000
E = 320000
D_FEAT = 128
D_LABEL = 16
H = 128
OUT = 16

NC = 2
NS = 16
B = 400
NB = 50
T = 50
ACC_ROWS = 10240
DUMMY = N
ZCH = 10
COPY_ROWS = ACC_ROWS // NS
QW = 32
NQ = 4
TCOPY = N // NS


def _spmm_body(tf0, tf1, tf2, tf3, tl0, tl1, tl2, tl3, srcs, dsts, zsrc,
               of10, of11, of12, of13, of20, of21, of22, of23,
               ol10, ol11, ol12, ol13, ol20, ol21, ol22, ol23,
               isrc, idst, buf0, buf1, zbuf, tbl, acc,
               sem0, sem1, ssem0, ssem1):
    cid = lax.axis_index("c")
    sid = lax.axis_index("s")
    pltpu.sync_copy(zsrc, zbuf)

    def run(tables, outs):
        for es in range(2):
            pltpu.sync_copy(srcs.at[es, pl.ds(sid * T, T)], isrc)
            pltpu.sync_copy(dsts.at[es, pl.ds(sid * T, T)], idst)
            for q in range(NQ):
                pltpu.sync_copy(tables[q].at[pl.ds(sid * TCOPY, TCOPY)],
                                tbl.at[pl.ds(sid * TCOPY, TCOPY)])
                base = 0
                for k in range(ZCH):
                    pltpu.sync_copy(
                        zbuf, acc.at[pl.ds(sid * (64 * ZCH) + k * 64, 64)])
                plsc.subcore_barrier()
                pltpu.async_copy(tbl.at[isrc.at[base + 0]], buf0, sem0)
                pltpu.async_copy(tbl.at[isrc.at[base + 1]], buf1, sem1)

                def it(jj, carry):
                    j = base + 2 * jj
                    pltpu.make_async_copy(tbl.at[isrc.at[j]], buf0, sem0).wait()
                    pltpu.async_copy(buf0, acc.at[idst.at[j]], ssem0, add=True)
                    pltpu.make_async_copy(tbl.at[isrc.at[j + 1]], buf1, sem1).wait()
                    pltpu.async_copy(buf1, acc.at[idst.at[j + 1]], ssem1, add=True)
                    pltpu.make_async_copy(buf0, acc.at[idst.at[j]], ssem0).wait()
                    pltpu.async_copy(tbl.at[isrc.at[j + 2]], buf0, sem0)
                    pltpu.make_async_copy(buf1, acc.at[idst.at[j + 1]], ssem1).wait()
                    pltpu.async_copy(tbl.at[isrc.at[j + 3]], buf1, sem1)
                    return carry

                lax.fori_loop(0, (T - 2) // 2, it, 0)
                pltpu.make_async_copy(tbl.at[isrc.at[base + T - 2]], buf0, sem0).wait()
                pltpu.async_copy(buf0, acc.at[idst.at[base + T - 2]], ssem0, add=True)
                pltpu.make_async_copy(tbl.at[isrc.at[base + T - 1]], buf1, sem1).wait()
                pltpu.async_copy(buf1, acc.at[idst.at[base + T - 1]], ssem1, add=True)
                pltpu.make_async_copy(buf0, acc.at[idst.at[base + T - 2]], ssem0).wait()
                pltpu.make_async_copy(buf1, acc.at[idst.at[base + T - 1]], ssem1).wait()
                plsc.subcore_barrier()
                pltpu.sync_copy(acc.at[pl.ds(sid * COPY_ROWS, COPY_ROWS)],
                                outs[es][q].at[pl.ds(sid * COPY_ROWS,
                                                     COPY_ROWS)])
            plsc.subcore_barrier()

    @pl.when(cid == 0)
    def _():
        run((tf0, tf1, tf2, tf3),
            ((of10, of11, of12, of13), (of20, of21, of22, of23)))

    @pl.when(cid == 1)
    def _():
        run((tl0, tl1, tl2, tl3),
            ((ol10, ol11, ol12, ol13), (ol20, ol21, ol22, ol23)))


def _make_spmm():
    if _SPMM_CACHE:
        return _SPMM_CACHE[0]
    f = pl.kernel(
        _spmm_body,
        out_type=[jax.ShapeDtypeStruct((ACC_ROWS, QW), jnp.float32)] * 16,
        mesh=plsc.VectorSubcoreMesh(core_axis_name="c", subcore_axis_name="s",
                                    num_cores=NC, num_subcores=NS),
        compiler_params=pltpu.CompilerParams(use_tc_tiling_on_sc=False),
        scratch_types=[
            pltpu.VMEM((T, B), jnp.int32),
            pltpu.VMEM((T, B), jnp.int32),
            pltpu.VMEM((B, QW), jnp.float32),
            pltpu.VMEM((B, QW), jnp.float32),
            pltpu.VMEM((64, QW), jnp.float32),
            pltpu.VMEM_SHARED((N, QW), jnp.float32),
            pltpu.VMEM_SHARED((ACC_ROWS, QW), jnp.float32),
            pltpu.SemaphoreType.DMA,
            pltpu.SemaphoreType.DMA,
            pltpu.SemaphoreType.DMA,
            pltpu.SemaphoreType.DMA,
        ],
    )
    _SPMM_CACHE.append(f)
    return f


_SPMM_CACHE = []


def _prep_edges(e):
    pad = NS * NB * B - E
    src = jnp.pad(e[0], (0, pad)).reshape(NS, NB, B)
    dst = jnp.pad(e[1], (0, pad), constant_values=DUMMY).reshape(NS, NB, B)
    src = jnp.concatenate([src, jnp.zeros((NS, T - NB, B), jnp.int32)], axis=1)
    dst = jnp.concatenate([dst, jnp.full((NS, T - NB, B), DUMMY, jnp.int32)],
                          axis=1)
    return src.reshape(NS * T, B), dst.reshape(NS * T, B)


RB = 1000


def _proj_body(xf, xl, wf, wl, h0f, h0l):
    h0f[...] = jnp.dot(xf[...], wf[...], preferred_element_type=jnp.float32)
    h0l[...] = jnp.dot(xl[...], wl[...], preferred_element_type=jnp.float32)


_proj = pl.pallas_call(
    _proj_body,
    grid=(N // RB,),
    in_specs=[
        pl.BlockSpec((RB, D_FEAT), lambda i: (i, 0)),
        pl.BlockSpec((RB, D_LABEL), lambda i: (i, 0)),
        pl.BlockSpec((D_FEAT, H), lambda i: (0, 0)),
        pl.BlockSpec((D_LABEL, H), lambda i: (0, 0)),
    ],
    out_specs=[
        pl.BlockSpec((RB, H), lambda i: (i, 0)),
        pl.BlockSpec((RB, H), lambda i: (i, 0)),
    ],
    out_shape=[jax.ShapeDtypeStruct((N, H), jnp.float32)] * 2,
)


def _gate_body(h0f, f10, f11, f12, f13, f20, f21, f22, f23,
               h0l, l10, l11, l12, l13, l20, l21, l22, l23,
               wg1, bg1, wg2r, bg2, wout, bout, out, gate):
    hf = jnp.concatenate(
        [h0f[...], f10[...], f11[...], f12[...], f13[...],
         f20[...], f21[...], f22[...], f23[...]], axis=1)
    hl = jnp.concatenate(
        [h0l[...], l10[...], l11[...], l12[...], l13[...],
         l20[...], l21[...], l22[...], l23[...]], axis=1)
    gi = jnp.concatenate([hf, hl], axis=1)
    t = jnp.maximum(
        jnp.dot(gi, wg1[...], preferred_element_type=jnp.float32) + bg1[...], 0.0)
    g = jax.nn.sigmoid(jnp.sum(t * wg2r[...], axis=1, keepdims=True) + bg2[...])
    fused = g * hf + (1.0 - g) * hl
    out[...] = jnp.dot(fused, wout[...], preferred_element_type=jnp.float32) + bout[...]
    gate[...] = jnp.broadcast_to(g, (RB, 3 * H))


_gate = pl.pallas_call(
    _gate_body,
    grid=(N // RB,),
    in_specs=[pl.BlockSpec((RB, H), lambda i: (i, 0))]
    + [pl.BlockSpec((RB, QW), lambda i: (i, 0))] * 8
    + [pl.BlockSpec((RB, H), lambda i: (i, 0))]
    + [pl.BlockSpec((RB, QW), lambda i: (i, 0))] * 8 + [
        pl.BlockSpec((6 * H, H), lambda i: (0, 0)),
        pl.BlockSpec((1, H), lambda i: (0, 0)),
        pl.BlockSpec((1, H), lambda i: (0, 0)),
        pl.BlockSpec((1, 1), lambda i: (0, 0)),
        pl.BlockSpec((3 * H, OUT), lambda i: (0, 0)),
        pl.BlockSpec((1, OUT), lambda i: (0, 0)),
    ],
    out_specs=[
        pl.BlockSpec((RB, OUT), lambda i: (i, 0)),
        pl.BlockSpec((RB, 3 * H), lambda i: (i, 0)),
    ],
    out_shape=[
        jax.ShapeDtypeStruct((N, OUT), jnp.float32),
        jax.ShapeDtypeStruct((N, 3 * H), jnp.float32),
    ],
)


def kernel(x_feat, x_label, edge_index1, edge_index2,
           W_feat, W_label, Wg1, bg1, Wg2, bg2, Wout, bout):
    h0f, h0l = _proj(x_feat, x_label, W_feat, W_label)
    s1, d1 = _prep_edges(edge_index1)
    s2, d2 = _prep_edges(edge_index2)
    srcs = jnp.stack([s1, s2])
    dsts = jnp.stack([d1, d2])
    fq = [h0f[:, i * QW:(i + 1) * QW] for i in range(NQ)]
    lq = [h0l[:, i * QW:(i + 1) * QW] for i in range(NQ)]
    outs = _make_spmm()(
        *fq, *lq, srcs, dsts, jnp.zeros((64, QW), jnp.float32))
    out, gate = _gate(
        h0f, *outs[0:8], h0l, *outs[8:16],
        Wg1, bg1.reshape(1, H), Wg2.reshape(1, H), bg2.reshape(1, 1),
        Wout, bout.reshape(1, OUT))
    return (out, gate)
